# baseline (speedup 1.0000x reference)
import functools

import jax
import jax.numpy as jnp
from jax import lax
from jax.experimental import pallas as pl
from jax.experimental.pallas import tpu as pltpu
from jax.experimental.pallas import tpu_sc as plsc

EXPERT_INTER = 1408
HIDDEN = 2048
TOP_K = 8

RB1 = 1408
SC_ROWS = 512
TC_ROWS = HIDDEN - SC_ROWS
NW = 32
RPW = SC_ROWS // NW
NCH = EXPERT_INTER // 16


def _inter_kernel(idx_ref, w_ref, x_ref, gate_ref, up_ref, o_ref):
    k = pl.program_id(0)
    g = jax.lax.dot_general(
        x_ref[...], gate_ref[0],
        (((1,), (1,)), ((), ())),
        preferred_element_type=jnp.float32,
    )
    u = jax.lax.dot_general(
        x_ref[...], up_ref[0],
        (((1,), (1,)), ((), ())),
        preferred_element_type=jnp.float32,
    )
    o_ref[0] = (g * jax.nn.sigmoid(g)) * u * w_ref[k]


def _down_kernel(idx_ref, w_ref, inter_ref, down_ref, o_ref):
    k = pl.program_id(1)
    part = jax.lax.dot_general(
        inter_ref[0], down_ref[0],
        (((1,), (1,)), ((), ())),
        preferred_element_type=jnp.float32,
    )

    @pl.when(k == 0)
    def _init():
        o_ref[...] = part

    @pl.when(k > 0)
    def _acc():
        o_ref[...] += part


def _sc_down_kernel(inter_hbm, rowidx_hbm, downflat_hbm, out_hbm,
                    inter_v, idx_v, buf_v, out_v, sem_a, sem_b, sem_s):
    wid = lax.axis_index("s") * 2 + lax.axis_index("c")

    pltpu.sync_copy(rowidx_hbm.at[pl.ds(wid * (TOP_K * RPW), TOP_K * RPW)],
                    idx_v)
    pltpu.sync_copy(inter_hbm, inter_v)

    sems = [sem_a, sem_b]
    copies = [None, None]
    idx0 = idx_v[pl.ds(0, RPW)]
    copies[0] = pltpu.async_copy(downflat_hbm.at[idx0], buf_v.at[0], sems[0])

    acc = [jnp.zeros((16,), jnp.float32) for _ in range(RPW)]
    for k in range(TOP_K):
        if k + 1 < TOP_K:
            idxn = idx_v[pl.ds((k + 1) * RPW, RPW)]
            copies[(k + 1) % 2] = pltpu.async_copy(
                downflat_hbm.at[idxn], buf_v.at[(k + 1) % 2], sems[(k + 1) % 2])
        copies[k % 2].wait()
        kb = k % 2

        def body(c, acc):
            iv = inter_v[k, pl.ds(c * 16, 16)]
            return tuple(
                acc[r] + buf_v[kb, r, pl.ds(c * 16, 16)] * iv
                for r in range(RPW)
            )

        acc = list(lax.fori_loop(0, NCH, body, tuple(acc)))

    lanes = lax.iota(jnp.int32, 16)
    dnums = lax.GatherDimensionNumbers(
        offset_dims=(), collapsed_slice_dims=(0,), start_index_map=(0,))
    outv = jnp.zeros((16,), jnp.float32)
    for r in range(RPW):
        v = acc[r]
        for s in (8, 4, 2, 1):
            perm = lax.gather(v, (lanes ^ s)[:, None], dnums, (1,),
                              mode=lax.GatherScatterMode.PROMISE_IN_BOUNDS)
            v = v + perm
        outv = jnp.where(lanes == r, v, outv)
    out_v[...] = outv
    pltpu.sync_copy(out_v, out_hbm.at[pl.ds(wid * RPW, RPW)])


def kernel(x_bc1t, topk_idx, topk_weights, gate_up_all, down_all):
    x = x_bc1t.reshape(1, HIDDEN)
    idx = topk_idx.astype(jnp.int32)
    nb1 = EXPERT_INTER // RB1

    inter = pl.pallas_call(
        _inter_kernel,
        grid_spec=pltpu.PrefetchScalarGridSpec(
            num_scalar_prefetch=2,
            grid=(TOP_K, nb1),
            in_specs=[
                pl.BlockSpec((1, HIDDEN), lambda k, b, idx, w: (0, 0)),
                pl.BlockSpec((1, RB1, HIDDEN),
                             lambda k, b, idx, w: (idx[k], b, 0)),
                pl.BlockSpec((1, RB1, HIDDEN),
                             lambda k, b, idx, w: (idx[k], b + EXPERT_INTER // RB1, 0)),
            ],
            out_specs=pl.BlockSpec((1, 1, RB1), lambda k, b, idx, w: (k, 0, b)),
        ),
        out_shape=jax.ShapeDtypeStruct((TOP_K, 1, EXPERT_INTER), jnp.float32),
    )(idx, topk_weights, x, gate_up_all, gate_up_all)

    out_tc = pl.pallas_call(
        _down_kernel,
        grid_spec=pltpu.PrefetchScalarGridSpec(
            num_scalar_prefetch=2,
            grid=(1, TOP_K),
            in_specs=[
                pl.BlockSpec((1, 1, EXPERT_INTER), lambda b, k, idx, w: (k, 0, 0)),
                pl.BlockSpec((1, TC_ROWS, EXPERT_INTER),
                             lambda b, k, idx, w: (idx[k], b, 0)),
            ],
            out_specs=pl.BlockSpec((1, TC_ROWS), lambda b, k, idx, w: (0, b)),
        ),
        out_shape=jax.ShapeDtypeStruct((1, TC_ROWS), jnp.float32),
    )(idx, topk_weights, inter, down_all[:, :TC_ROWS, :])

    base = idx * HIDDEN + TC_ROWS
    rows = jnp.arange(RPW, dtype=jnp.int32)
    woff = jnp.arange(NW, dtype=jnp.int32) * RPW
    rowidx = (base[None, :, None] + woff[:, None, None] + rows[None, None, :])
    rowidx = rowidx.reshape(-1)

    inter2d = inter.reshape(TOP_K, EXPERT_INTER)
    downflat = down_all.reshape(60 * HIDDEN, EXPERT_INTER)

    sc_call = functools.partial(
        pl.kernel,
        mesh=plsc.VectorSubcoreMesh(core_axis_name="c", subcore_axis_name="s"),
        out_type=jax.ShapeDtypeStruct((SC_ROWS,), jnp.float32),
        scratch_types=[
            pltpu.VMEM((TOP_K, EXPERT_INTER), jnp.float32),
            pltpu.VMEM((TOP_K * RPW,), jnp.int32),
            pltpu.VMEM((2, RPW, EXPERT_INTER), jnp.float32),
            pltpu.VMEM((RPW,), jnp.float32),
            pltpu.SemaphoreType.DMA,
            pltpu.SemaphoreType.DMA,
            pltpu.SemaphoreType.DMA,
        ],
    )(_sc_down_kernel)
    out_sc = sc_call(inter2d, rowidx, downflat)

    out = jnp.concatenate([out_tc.reshape(TC_ROWS), out_sc], axis=0)
    return out.reshape(1, HIDDEN, 1, 1)


# speedup vs baseline: 3.6485x; 3.6485x over previous
import functools

import jax
import jax.numpy as jnp
from jax import lax
from jax.experimental import pallas as pl
from jax.experimental.pallas import tpu as pltpu
from jax.experimental.pallas import tpu_sc as plsc

EXPERT_INTER = 1408
HIDDEN = 2048
TOP_K = 8

RB1 = 1408
SC_ROWS = 512
TC_ROWS = HIDDEN - SC_ROWS
NW = 32
RPW = SC_ROWS // NW
NCH = EXPERT_INTER // 16


def _inter_kernel(idx_ref, w_ref, x_ref, gate_ref, up_ref, o_ref):
    k = pl.program_id(0)
    g = jax.lax.dot_general(
        x_ref[...], gate_ref[0],
        (((1,), (1,)), ((), ())),
        preferred_element_type=jnp.float32,
    )
    u = jax.lax.dot_general(
        x_ref[...], up_ref[0],
        (((1,), (1,)), ((), ())),
        preferred_element_type=jnp.float32,
    )
    o_ref[0] = (g * jax.nn.sigmoid(g)) * u * w_ref[k]


def _down_kernel(idx_ref, w_ref, inter_ref, down_ref, o_ref):
    k = pl.program_id(1)
    part = jax.lax.dot_general(
        inter_ref[0], down_ref[0],
        (((1,), (1,)), ((), ())),
        preferred_element_type=jnp.float32,
    )

    @pl.when(k == 0)
    def _init():
        o_ref[...] = part

    @pl.when(k > 0)
    def _acc():
        o_ref[...] += part


def _sc_down_kernel(inter_hbm, rowidx_hbm, downflat_hbm, out_hbm,
                    inter_v, idx_v, buf_v, out_v, sem_a, sem_b, sem_s):
    wid = lax.axis_index("s") * 2 + lax.axis_index("c")

    pltpu.sync_copy(rowidx_hbm.at[pl.ds(wid * (TOP_K * RPW), TOP_K * RPW)],
                    idx_v)
    pltpu.sync_copy(inter_hbm, inter_v)

    sems = [sem_a, sem_b]
    copies = [None, None]
    idx0 = idx_v[pl.ds(0, RPW)]
    copies[0] = pltpu.async_copy(downflat_hbm.at[idx0], buf_v.at[0], sems[0])

    acc = [jnp.zeros((16,), jnp.float32) for _ in range(RPW)]
    for k in range(TOP_K):
        if k + 1 < TOP_K:
            idxn = idx_v[pl.ds((k + 1) * RPW, RPW)]
            copies[(k + 1) % 2] = pltpu.async_copy(
                downflat_hbm.at[idxn], buf_v.at[(k + 1) % 2], sems[(k + 1) % 2])
        copies[k % 2].wait()
        kb = k % 2

        def body(c, acc):
            iv = inter_v[k, pl.ds(c * 16, 16)]
            return tuple(
                acc[r] + buf_v[kb, r, pl.ds(c * 16, 16)] * iv
                for r in range(RPW)
            )

        acc = list(lax.fori_loop(0, NCH, body, tuple(acc)))

    lanes = lax.iota(jnp.int32, 16)
    dnums = lax.GatherDimensionNumbers(
        offset_dims=(), collapsed_slice_dims=(0,), start_index_map=(0,))
    outv = jnp.zeros((16,), jnp.float32)
    for r in range(RPW):
        v = acc[r]
        for s in (8, 4, 2, 1):
            perm = lax.gather(v, (lanes ^ s)[:, None], dnums, (1,),
                              mode=lax.GatherScatterMode.PROMISE_IN_BOUNDS)
            v = v + perm
        outv = jnp.where(lanes == r, v, outv)
    out_v[...] = outv
    pltpu.sync_copy(out_v, out_hbm.at[pl.ds(wid * RPW, RPW)])


def kernel(x_bc1t, topk_idx, topk_weights, gate_up_all, down_all):
    x = x_bc1t.reshape(1, HIDDEN)
    idx = topk_idx.astype(jnp.int32)
    nb1 = EXPERT_INTER // RB1

    inter = pl.pallas_call(
        _inter_kernel,
        grid_spec=pltpu.PrefetchScalarGridSpec(
            num_scalar_prefetch=2,
            grid=(TOP_K, nb1),
            in_specs=[
                pl.BlockSpec((1, HIDDEN), lambda k, b, idx, w: (0, 0)),
                pl.BlockSpec((1, RB1, HIDDEN),
                             lambda k, b, idx, w: (idx[k], b, 0)),
                pl.BlockSpec((1, RB1, HIDDEN),
                             lambda k, b, idx, w: (idx[k], b + EXPERT_INTER // RB1, 0)),
            ],
            out_specs=pl.BlockSpec((1, 1, RB1), lambda k, b, idx, w: (k, 0, b)),
        ),
        out_shape=jax.ShapeDtypeStruct((TOP_K, 1, EXPERT_INTER), jnp.float32),
    )(idx, topk_weights, x, gate_up_all, gate_up_all)

    rb2 = 512
    out_tc = pl.pallas_call(
        _down_kernel,
        grid_spec=pltpu.PrefetchScalarGridSpec(
            num_scalar_prefetch=2,
            grid=(TC_ROWS // rb2, TOP_K),
            in_specs=[
                pl.BlockSpec((1, 1, EXPERT_INTER), lambda b, k, idx, w: (k, 0, 0)),
                pl.BlockSpec((1, rb2, EXPERT_INTER),
                             lambda b, k, idx, w: (idx[k], b, 0)),
            ],
            out_specs=pl.BlockSpec((1, rb2), lambda b, k, idx, w: (0, b)),
        ),
        out_shape=jax.ShapeDtypeStruct((1, TC_ROWS), jnp.float32),
    )(idx, topk_weights, inter, down_all)

    base = idx * HIDDEN + TC_ROWS
    rows = jnp.arange(RPW, dtype=jnp.int32)
    woff = jnp.arange(NW, dtype=jnp.int32) * RPW
    rowidx = (base[None, :, None] + woff[:, None, None] + rows[None, None, :])
    rowidx = rowidx.reshape(-1)

    inter2d = inter.reshape(TOP_K, EXPERT_INTER)
    downflat = down_all.reshape(60 * HIDDEN, EXPERT_INTER)

    sc_call = functools.partial(
        pl.kernel,
        mesh=plsc.VectorSubcoreMesh(core_axis_name="c", subcore_axis_name="s"),
        out_type=jax.ShapeDtypeStruct((SC_ROWS,), jnp.float32),
        scratch_types=[
            pltpu.VMEM((TOP_K, EXPERT_INTER), jnp.float32),
            pltpu.VMEM((TOP_K * RPW,), jnp.int32),
            pltpu.VMEM((2, RPW, EXPERT_INTER), jnp.float32),
            pltpu.VMEM((RPW,), jnp.float32),
            pltpu.SemaphoreType.DMA,
            pltpu.SemaphoreType.DMA,
            pltpu.SemaphoreType.DMA,
        ],
    )(_sc_down_kernel)
    out_sc = sc_call(inter2d, rowidx, downflat)

    out = jnp.concatenate([out_tc.reshape(TC_ROWS), out_sc], axis=0)
    return out.reshape(1, HIDDEN, 1, 1)


# speedup vs baseline: 3.6513x; 1.0008x over previous
import functools

import jax
import jax.numpy as jnp
from jax import lax
from jax.experimental import pallas as pl
from jax.experimental.pallas import tpu as pltpu
from jax.experimental.pallas import tpu_sc as plsc

EXPERT_INTER = 1408
HIDDEN = 2048
TOP_K = 8

RB1 = 1408
SC_ROWS = 512
TC_ROWS = HIDDEN - SC_ROWS
NW = 32
RPW = SC_ROWS // NW
NCH = EXPERT_INTER // 16


def _inter_kernel(idx_ref, w_ref, x_ref, gate_ref, up_ref, o_ref):
    k = pl.program_id(0)
    g = jax.lax.dot_general(
        x_ref[...], gate_ref[0],
        (((1,), (1,)), ((), ())),
        preferred_element_type=jnp.float32,
    )
    u = jax.lax.dot_general(
        x_ref[...], up_ref[0],
        (((1,), (1,)), ((), ())),
        preferred_element_type=jnp.float32,
    )
    o_ref[0] = (g * jax.nn.sigmoid(g)) * u * w_ref[k]


def _down_kernel(idx_ref, w_ref, inter_ref, down_ref, o_ref):
    k = pl.program_id(1)
    part = jax.lax.dot_general(
        inter_ref[0], down_ref[0],
        (((1,), (1,)), ((), ())),
        preferred_element_type=jnp.float32,
    )

    @pl.when(k == 0)
    def _init():
        o_ref[...] = part

    @pl.when(k > 0)
    def _acc():
        o_ref[...] += part


def _sc_down_kernel(inter_hbm, rowidx_hbm, downflat_hbm, out_hbm,
                    inter_v, idx_v, buf_v, out_v, sem_a, sem_b, sem_s):
    wid = lax.axis_index("s") * 2 + lax.axis_index("c")

    pltpu.sync_copy(rowidx_hbm.at[pl.ds(wid * (TOP_K * RPW), TOP_K * RPW)],
                    idx_v)
    pltpu.sync_copy(inter_hbm, inter_v)

    sems = [sem_a, sem_b]
    copies = [None, None]
    idx0 = idx_v[pl.ds(0, RPW)]
    copies[0] = pltpu.async_copy(downflat_hbm.at[idx0], buf_v.at[0], sems[0])

    acc = [jnp.zeros((16,), jnp.float32) for _ in range(RPW)]
    for k in range(TOP_K):
        if k + 1 < TOP_K:
            idxn = idx_v[pl.ds((k + 1) * RPW, RPW)]
            copies[(k + 1) % 2] = pltpu.async_copy(
                downflat_hbm.at[idxn], buf_v.at[(k + 1) % 2], sems[(k + 1) % 2])
        copies[k % 2].wait()
        kb = k % 2

        def body(c, acc):
            iv = inter_v[k, pl.ds(c * 16, 16)]
            return tuple(
                acc[r] + buf_v[kb, r, pl.ds(c * 16, 16)] * iv
                for r in range(RPW)
            )

        acc = list(lax.fori_loop(0, NCH, body, tuple(acc)))

    lanes = lax.iota(jnp.int32, 16)
    dnums = lax.GatherDimensionNumbers(
        offset_dims=(), collapsed_slice_dims=(0,), start_index_map=(0,))
    outv = jnp.zeros((16,), jnp.float32)
    for r in range(RPW):
        v = acc[r]
        for s in (8, 4, 2, 1):
            perm = lax.gather(v, (lanes ^ s)[:, None], dnums, (1,),
                              mode=lax.GatherScatterMode.PROMISE_IN_BOUNDS)
            v = v + perm
        outv = jnp.where(lanes == r, v, outv)
    out_v[...] = outv
    pltpu.sync_copy(out_v, out_hbm.at[pl.ds(wid * RPW, RPW)])


def kernel(x_bc1t, topk_idx, topk_weights, gate_up_all, down_all):
    x = x_bc1t.reshape(1, HIDDEN)
    idx = topk_idx.astype(jnp.int32)
    nb1 = EXPERT_INTER // RB1

    inter = pl.pallas_call(
        _inter_kernel,
        grid_spec=pltpu.PrefetchScalarGridSpec(
            num_scalar_prefetch=2,
            grid=(TOP_K, nb1),
            in_specs=[
                pl.BlockSpec((1, HIDDEN), lambda k, b, idx, w: (0, 0)),
                pl.BlockSpec((1, RB1, HIDDEN),
                             lambda k, b, idx, w: (idx[k], b, 0)),
                pl.BlockSpec((1, RB1, HIDDEN),
                             lambda k, b, idx, w: (idx[k], b + EXPERT_INTER // RB1, 0)),
            ],
            out_specs=pl.BlockSpec((1, 1, RB1), lambda k, b, idx, w: (k, 0, b)),
        ),
        out_shape=jax.ShapeDtypeStruct((TOP_K, 1, EXPERT_INTER), jnp.float32),
    )(idx, topk_weights, x, gate_up_all, gate_up_all)

    base = idx * HIDDEN + TC_ROWS
    rows = jnp.arange(RPW, dtype=jnp.int32)
    woff = jnp.arange(NW, dtype=jnp.int32) * RPW
    rowidx = (base[None, :, None] + woff[:, None, None] + rows[None, None, :])
    rowidx = rowidx.reshape(-1)

    inter2d = inter.reshape(TOP_K, EXPERT_INTER)
    downflat = down_all.reshape(60 * HIDDEN, EXPERT_INTER)

    sc_call = functools.partial(
        pl.kernel,
        mesh=plsc.VectorSubcoreMesh(core_axis_name="c", subcore_axis_name="s"),
        out_type=jax.ShapeDtypeStruct((SC_ROWS,), jnp.float32),
        scratch_types=[
            pltpu.VMEM((TOP_K, EXPERT_INTER), jnp.float32),
            pltpu.VMEM((TOP_K * RPW,), jnp.int32),
            pltpu.VMEM((2, RPW, EXPERT_INTER), jnp.float32),
            pltpu.VMEM((RPW,), jnp.float32),
            pltpu.SemaphoreType.DMA,
            pltpu.SemaphoreType.DMA,
            pltpu.SemaphoreType.DMA,
        ],
    )(_sc_down_kernel)
    out_sc = sc_call(inter2d, rowidx, downflat)

    rb2 = 512
    out_tc = pl.pallas_call(
        _down_kernel,
        grid_spec=pltpu.PrefetchScalarGridSpec(
            num_scalar_prefetch=2,
            grid=(TC_ROWS // rb2, TOP_K),
            in_specs=[
                pl.BlockSpec((1, 1, EXPERT_INTER), lambda b, k, idx, w: (k, 0, 0)),
                pl.BlockSpec((1, rb2, EXPERT_INTER),
                             lambda b, k, idx, w: (idx[k], b, 0)),
            ],
            out_specs=pl.BlockSpec((1, rb2), lambda b, k, idx, w: (0, b)),
        ),
        out_shape=jax.ShapeDtypeStruct((1, TC_ROWS), jnp.float32),
    )(idx, topk_weights, inter, down_all)

    out = jnp.concatenate([out_tc.reshape(TC_ROWS), out_sc], axis=0)
    return out.reshape(1, HIDDEN, 1, 1)


# speedup vs baseline: 4.1605x; 1.1395x over previous
import jax
import jax.numpy as jnp
from jax.experimental import pallas as pl
from jax.experimental.pallas import tpu as pltpu

EXPERT_INTER = 1408
HIDDEN = 2048
TOP_K = 8

HB1 = EXPERT_INTER // 2
HB2 = HIDDEN // 2


def _fused_kernel(idx_ref, w_ref, x_ref, gate_ref, up_ref, down_ref, o_ref,
                  inter_s, out_s):
    k = pl.program_id(0)
    s = pl.program_id(1)

    @pl.when(jnp.logical_and(k == 0, s == 0))
    def _zero():
        out_s[...] = jnp.zeros_like(out_s)

    @pl.when(s < 2)
    def _phase1():
        g = jax.lax.dot_general(
            x_ref[...], gate_ref[0],
            (((1,), (1,)), ((), ())),
            preferred_element_type=jnp.float32,
        )
        u = jax.lax.dot_general(
            x_ref[...], up_ref[0],
            (((1,), (1,)), ((), ())),
            preferred_element_type=jnp.float32,
        )
        val = (g * jax.nn.sigmoid(g)) * u * w_ref[k]

        @pl.when(s == 0)
        def _lo():
            inter_s[:, 0:HB1] = val

        @pl.when(s == 1)
        def _hi():
            inter_s[:, HB1:EXPERT_INTER] = val

    @pl.when(s >= 2)
    def _phase2():
        part = jax.lax.dot_general(
            inter_s[...], down_ref[0],
            (((1,), (1,)), ((), ())),
            preferred_element_type=jnp.float32,
        )

        @pl.when(s == 2)
        def _lo():
            out_s[:, 0:HB2] += part

        @pl.when(s == 3)
        def _hi():
            out_s[:, HB2:HIDDEN] += part

    @pl.when(jnp.logical_and(k == TOP_K - 1, s == 3))
    def _emit():
        o_ref[...] = out_s[...]


def kernel(x_bc1t, topk_idx, topk_weights, gate_up_all, down_all):
    x = x_bc1t.reshape(1, HIDDEN)
    idx = topk_idx.astype(jnp.int32)

    out = pl.pallas_call(
        _fused_kernel,
        grid_spec=pltpu.PrefetchScalarGridSpec(
            num_scalar_prefetch=2,
            grid=(TOP_K, 4),
            in_specs=[
                pl.BlockSpec((1, HIDDEN), lambda k, s, idx, w: (0, 0)),
                pl.BlockSpec(
                    (1, HB1, HIDDEN),
                    lambda k, s, idx, w: (idx[k], jnp.minimum(s, 1), 0)),
                pl.BlockSpec(
                    (1, HB1, HIDDEN),
                    lambda k, s, idx, w: (idx[k], jnp.minimum(s, 1) + 2, 0)),
                pl.BlockSpec(
                    (1, HB2, EXPERT_INTER),
                    lambda k, s, idx, w: (idx[k], jnp.maximum(s - 2, 0), 0)),
            ],
            out_specs=pl.BlockSpec((1, HIDDEN), lambda k, s, idx, w: (0, 0)),
            scratch_shapes=[
                pltpu.VMEM((1, EXPERT_INTER), jnp.float32),
                pltpu.VMEM((1, HIDDEN), jnp.float32),
            ],
        ),
        out_shape=jax.ShapeDtypeStruct((1, HIDDEN), jnp.float32),
    )(idx, topk_weights, x, gate_up_all, gate_up_all, down_all)

    return out.reshape(1, HIDDEN, 1, 1)
